# Initial kernel scaffold; baseline (speedup 1.0000x reference)
import jax
import jax.numpy as jnp
from jax.experimental import pallas as pl


def kernel(edge_index, x, W1, b1, W2, b2):
    raise NotImplementedError("write your pallas kernel here")


# speedup vs baseline: 9.1282x; 9.1282x over previous
import functools

import jax
import jax.numpy as jnp
from jax import lax
from jax.experimental import pallas as pl
from jax.experimental.pallas import tpu as pltpu
from jax.experimental.pallas import tpu_sc as plsc

N = 10000
D = 128
E = 320000

NC = 2
NS = 16
NW = NC * NS

NPAD = 10240
NBLK = NPAD // 128
CH = 128
EPT_CH = 80
NHALF = 2
CH_H = EPT_CH // NHALF
EPT = CH * EPT_CH
EPAD = NW * EPT
ROWS_PT = NPAD // NS
E_PT = E // NW

_mesh = plsc.VectorSubcoreMesh(
    core_axis_name="c", subcore_axis_name="s", num_cores=NC, num_subcores=NS)


def _zero_f32_ref(ref, nvec):
  z = jnp.zeros((16,), jnp.float32)

  def body(i, _):
    ref[pl.ds(i * 16, 16)] = z
    return 0

  lax.fori_loop(0, nvec, body, 0)


def _zero_f32_ref2d(ref, nrows, ncols):
  z = jnp.zeros((16,), jnp.float32)

  def body(i, _):
    for j in range(ncols // 16):
      ref[i, pl.ds(j * 16, 16)] = z
    return 0

  lax.fori_loop(0, nrows, body, 0)


@functools.partial(
    pl.kernel,
    out_type=jax.ShapeDtypeStruct((NW, NPAD), jnp.float32),
    mesh=_mesh,
    scratch_types=[
        pltpu.VMEM((E_PT,), jnp.int32),
        pltpu.VMEM((NPAD,), jnp.float32),
    ],
    compiler_params=pltpu.CompilerParams(needs_layout_passes=False),
)
def _sc_deg(dst_hbm, out_hbm, dstv, degv):
  wid = lax.axis_index("c") * NS + lax.axis_index("s")
  pltpu.sync_copy(dst_hbm.at[wid], dstv)
  _zero_f32_ref(degv, NPAD // 16)
  ones = jnp.ones((16,), jnp.float32)

  def body(i, _):
    idx = dstv[pl.ds(i * 16, 16)]
    plsc.addupdate_scatter(degv, [idx], ones)
    return 0

  lax.fori_loop(0, E_PT // 16, body, 0)
  pltpu.sync_copy(degv, out_hbm.at[wid])


@functools.partial(
    pl.kernel,
    out_type=jax.ShapeDtypeStruct((NC, NPAD, D), jnp.float32),
    mesh=_mesh,
    scratch_types=[
        pltpu.VMEM((CH_H, CH), jnp.int32),
        pltpu.VMEM((CH_H, CH), jnp.int32),
        pltpu.VMEM((CH, D), jnp.float32),
        pltpu.VMEM((CH, D), jnp.float32),
        pltpu.VMEM_SHARED((NPAD, D), jnp.float32),
        pltpu.SemaphoreType.DMA,
        pltpu.SemaphoreType.DMA,
    ],
)
def _sc_edges(y_hbm, src_hbm, dst_hbm, out_hbm,
              srcv, dstv, rows0, rows1, acc, sem0, sem1):
  cid = lax.axis_index("c")
  sid = lax.axis_index("s")
  wid = cid * NS + sid
  rows = (rows0, rows1)
  sems = (sem0, sem1)

  _zero_f32_ref2d(rows0, CH, D)
  for t in range(ROWS_PT // CH):
    pltpu.sync_copy(rows0, acc.at[pl.ds(sid * ROWS_PT + t * CH, CH)])
  plsc.subcore_barrier()

  for half in range(NHALF):
    pltpu.sync_copy(src_hbm.at[wid, half], srcv)
    pltpu.sync_copy(dst_hbm.at[wid, half], dstv)
    pltpu.async_copy(y_hbm.at[srcv.at[0]], rows0, sem0)

    def outer(k2, _):
      for b in range(2):
        k = k2 * 2 + b

        @pl.when(k + 1 < CH_H)
        def _start():
          pltpu.async_copy(y_hbm.at[srcv.at[k + 1]], rows[1 - b], sems[1 - b])

        pltpu.make_async_copy(y_hbm.at[srcv.at[k]], rows[b], sems[b]).wait()
        pltpu.sync_copy(rows[b], acc.at[dstv.at[k]], add=True)
      return 0

    lax.fori_loop(0, CH_H // 2, outer, 0)

  plsc.subcore_barrier()
  pltpu.sync_copy(acc.at[pl.ds(sid * ROWS_PT, ROWS_PT)],
                  out_hbm.at[cid, pl.ds(sid * ROWS_PT, ROWS_PT)])


def _dot(a, b):
  return lax.dot_general(a, b, (((1,), (0,)), ((), ())),
                         precision=lax.Precision.HIGHEST,
                         preferred_element_type=jnp.float32)


def _tc_prep_body(x_ref, degt_ref, w_ref, y_ref, dinv_ref):
  deg = jnp.sum(degt_ref[...], axis=1, keepdims=True) + 1.0
  dinv = lax.rsqrt(deg)
  y_ref[...] = _dot(x_ref[...], w_ref[...]) * dinv
  dinv_ref[...] = dinv


def _tc_mid_body(p0_ref, p1_ref, y_ref, dinv_ref, b_ref, w_ref, out_ref):
  dinv = dinv_ref[...]
  h = (p0_ref[...] + p1_ref[...] + y_ref[...]) * dinv + b_ref[...]
  h = jnp.maximum(h, 0.0)
  out_ref[...] = _dot(h, w_ref[...]) * dinv


def _tc_final_body(p0_ref, p1_ref, y_ref, dinv_ref, b_ref, out_ref):
  h = (p0_ref[...] + p1_ref[...] + y_ref[...]) * dinv_ref[...] + b_ref[...]
  out_ref[...] = jnp.maximum(h, 0.0)


_row_spec = pl.BlockSpec((128, D), lambda i: (i, 0))
_col_spec = pl.BlockSpec((128, 1), lambda i: (i, 0))
_bias_spec = pl.BlockSpec((1, D), lambda i: (0, 0))
_w_spec = pl.BlockSpec((D, D), lambda i: (0, 0))

_tc_prep = pl.pallas_call(
    _tc_prep_body,
    grid=(NBLK,),
    in_specs=[_row_spec, pl.BlockSpec((128, NW), lambda i: (i, 0)), _w_spec],
    out_specs=[_row_spec, _col_spec],
    out_shape=[jax.ShapeDtypeStruct((NPAD, D), jnp.float32),
               jax.ShapeDtypeStruct((NPAD, 1), jnp.float32)],
)

_tc_mid = pl.pallas_call(
    _tc_mid_body,
    grid=(NBLK,),
    in_specs=[_row_spec, _row_spec, _row_spec, _col_spec, _bias_spec, _w_spec],
    out_specs=_row_spec,
    out_shape=jax.ShapeDtypeStruct((NPAD, D), jnp.float32),
)

_tc_final = pl.pallas_call(
    _tc_final_body,
    grid=(NBLK,),
    in_specs=[_row_spec, _row_spec, _row_spec, _col_spec, _bias_spec],
    out_specs=_row_spec,
    out_shape=jax.ShapeDtypeStruct((NPAD, D), jnp.float32),
)


def kernel(edge_index, x, W1, b1, W2, b2):
  src = edge_index[0]
  dst = edge_index[1]
  pad = jnp.full((EPAD - E,), N, dtype=jnp.int32)
  src3 = jnp.concatenate([src, pad]).reshape(NW, NHALF, CH_H, CH)
  dst3 = jnp.concatenate([dst, pad]).reshape(NW, NHALF, CH_H, CH)
  dst_deg = dst.reshape(NW, E_PT)
  xpad = jnp.zeros((NPAD, D), jnp.float32).at[:N].set(x)
  b1r = b1.reshape(1, D)
  b2r = b2.reshape(1, D)

  deg_parts = _sc_deg(dst_deg)
  degt = deg_parts.T
  y1, dinv = _tc_prep(xpad, degt, W1)
  part1 = _sc_edges(y1, src3, dst3)
  y2 = _tc_mid(part1[0], part1[1], y1, dinv, b1r, W2)
  part2 = _sc_edges(y2, src3, dst3)
  out = _tc_final(part2[0], part2[1], y2, dinv, b2r)
  return out[:N]


# speedup vs baseline: 9.9496x; 1.0900x over previous
import functools

import jax
import jax.numpy as jnp
from jax import lax
from jax.experimental import pallas as pl
from jax.experimental.pallas import tpu as pltpu
from jax.experimental.pallas import tpu_sc as plsc

N = 10000
D = 128
E = 320000

NC = 2
NS = 16
NW = NC * NS

NPAD = 10240
NBLK = NPAD // 128
CH = 128
EPT_CH = 80
NHALF = 2
CH_H = EPT_CH // NHALF
EPT = CH * EPT_CH
EPAD = NW * EPT
ROWS_PT = NPAD // NS
E_PT = E // NW

_mesh = plsc.VectorSubcoreMesh(
    core_axis_name="c", subcore_axis_name="s", num_cores=NC, num_subcores=NS)


def _zero_f32_ref(ref, nvec):
  z = jnp.zeros((16,), jnp.float32)

  def body(i, _):
    ref[pl.ds(i * 16, 16)] = z
    return 0

  lax.fori_loop(0, nvec, body, 0)


def _zero_f32_ref2d(ref, nrows, ncols):
  z = jnp.zeros((16,), jnp.float32)

  def body(i, _):
    for j in range(ncols // 16):
      ref[i, pl.ds(j * 16, 16)] = z
    return 0

  lax.fori_loop(0, nrows, body, 0)


E_PT16 = E // NS
RB_PT = 8
RB_TILES = NBLK // RB_PT


def _rsqrt_f32(d):
  i = plsc.bitcast(d, jnp.int32)
  i = jnp.int32(0x5F3759DF) - lax.shift_right_arithmetic(i, 1)
  h = plsc.bitcast(i, jnp.float32)
  for _ in range(3):
    h = h * (1.5 - 0.5 * d * h * h)
  return h


@functools.partial(
    pl.kernel,
    out_type=jax.ShapeDtypeStruct((NBLK, 128), jnp.float32),
    mesh=_mesh,
    scratch_types=[
        pltpu.VMEM((E_PT16,), jnp.int32),
        pltpu.VMEM((NBLK, 128), jnp.float32),
        pltpu.VMEM((RB_PT, 128), jnp.float32),
        pltpu.VMEM((RB_PT, 128), jnp.float32),
        pltpu.VMEM((NBLK,), jnp.int32),
        pltpu.VMEM_SHARED((NBLK, 128), jnp.float32),
    ],
    compiler_params=pltpu.CompilerParams(needs_layout_passes=False),
)
def _sc_deg(dst_hbm, dinv_hbm, dstv, degv, rowa, rowb, idxv, deg_sh):
  cid = lax.axis_index("c")
  sid = lax.axis_index("s")

  @pl.when(cid == 0)
  def _core0():
    pltpu.sync_copy(dst_hbm.at[sid], dstv)
    _zero_f32_ref2d(degv, NBLK, 128)
    for j in range(NBLK // 16):
      idxv[pl.ds(j * 16, 16)] = lax.iota(jnp.int32, 16) + j * 16

    @pl.when(sid < RB_TILES)
    def _zero_shared():
      _zero_f32_ref2d(rowa, RB_PT, 128)
      pltpu.sync_copy(rowa, deg_sh.at[pl.ds(sid * RB_PT, RB_PT)])

    plsc.subcore_barrier()
    ones = jnp.ones((16,), jnp.float32)

    def body(i, _):
      idx = dstv[pl.ds(i * 16, 16)]
      r = lax.shift_right_logical(idx, 7)
      c = lax.bitwise_and(idx, 127)
      plsc.addupdate_scatter(degv, [r, c], ones)
      return 0

    lax.fori_loop(0, E_PT16 // 16, body, 0)
    pltpu.sync_copy(degv, deg_sh.at[idxv], add=True)
    plsc.subcore_barrier()

    @pl.when(sid < RB_TILES)
    def _rsqrt_out():
      pltpu.sync_copy(deg_sh.at[pl.ds(sid * RB_PT, RB_PT)], rowa)
      for r in range(RB_PT):
        for c in range(128 // 16):
          d = rowa[r, pl.ds(c * 16, 16)] + 1.0
          rowb[r, pl.ds(c * 16, 16)] = _rsqrt_f32(d)
      pltpu.sync_copy(rowb, dinv_hbm.at[pl.ds(sid * RB_PT, RB_PT)])


@functools.partial(
    pl.kernel,
    out_type=[jax.ShapeDtypeStruct((NPAD, D), jnp.float32),
              jax.ShapeDtypeStruct((NPAD, D), jnp.float32)],
    mesh=_mesh,
    scratch_types=[
        pltpu.VMEM((CH_H, CH), jnp.int32),
        pltpu.VMEM((CH_H, CH), jnp.int32),
        pltpu.VMEM((CH, D), jnp.float32),
        pltpu.VMEM((CH, D), jnp.float32),
        pltpu.VMEM_SHARED((NPAD, D), jnp.float32),
        pltpu.SemaphoreType.DMA,
        pltpu.SemaphoreType.DMA,
    ],
)
def _sc_edges(y_hbm, src_hbm, dst_hbm, out0_hbm, out1_hbm,
              srcv, dstv, rows0, rows1, acc, sem0, sem1):
  cid = lax.axis_index("c")
  sid = lax.axis_index("s")
  wid = cid * NS + sid
  rows = (rows0, rows1)
  sems = (sem0, sem1)

  _zero_f32_ref2d(rows0, CH, D)
  for t in range(ROWS_PT // CH):
    pltpu.sync_copy(rows0, acc.at[pl.ds(sid * ROWS_PT + t * CH, CH)])
  plsc.subcore_barrier()

  for half in range(NHALF):
    pltpu.sync_copy(src_hbm.at[wid, half], srcv)
    pltpu.sync_copy(dst_hbm.at[wid, half], dstv)
    pltpu.async_copy(y_hbm.at[srcv.at[0]], rows0, sem0)

    def outer(k2, _):
      for b in range(2):
        k = k2 * 2 + b

        @pl.when(k + 1 < CH_H)
        def _start():
          pltpu.async_copy(y_hbm.at[srcv.at[k + 1]], rows[1 - b], sems[1 - b])

        pltpu.make_async_copy(y_hbm.at[srcv.at[k]], rows[b], sems[b]).wait()
        pltpu.sync_copy(rows[b], acc.at[dstv.at[k]], add=True)
      return 0

    lax.fori_loop(0, CH_H // 2, outer, 0)

  plsc.subcore_barrier()

  @pl.when(cid == 0)
  def _pub0():
    pltpu.sync_copy(acc.at[pl.ds(sid * ROWS_PT, ROWS_PT)],
                    out0_hbm.at[pl.ds(sid * ROWS_PT, ROWS_PT)])

  @pl.when(cid == 1)
  def _pub1():
    pltpu.sync_copy(acc.at[pl.ds(sid * ROWS_PT, ROWS_PT)],
                    out1_hbm.at[pl.ds(sid * ROWS_PT, ROWS_PT)])


def _dot(a, b):
  return lax.dot_general(a, b, (((1,), (0,)), ((), ())),
                         precision=lax.Precision.HIGHEST,
                         preferred_element_type=jnp.float32)


def _tc_prep_body(x_ref, dinv_ref, w_ref, y_ref):
  y_ref[...] = _dot(x_ref[...], w_ref[...]) * dinv_ref[...]


def _tc_mid_body(p0_ref, p1_ref, y_ref, dinv_ref, b_ref, w_ref, out_ref):
  dinv = dinv_ref[...]
  h = (p0_ref[...] + p1_ref[...] + y_ref[...]) * dinv + b_ref[...]
  h = jnp.maximum(h, 0.0)
  out_ref[...] = _dot(h, w_ref[...]) * dinv


def _tc_final_body(p0_ref, p1_ref, y_ref, dinv_ref, b_ref, out_ref):
  h = (p0_ref[...] + p1_ref[...] + y_ref[...]) * dinv_ref[...] + b_ref[...]
  out_ref[...] = jnp.maximum(h, 0.0)


TB = 512
TGRID = NPAD // TB
_row_spec = pl.BlockSpec((TB, D), lambda i: (i, 0))
_col_spec = pl.BlockSpec((TB, 1), lambda i: (i, 0))
_bias_spec = pl.BlockSpec((1, D), lambda i: (0, 0))
_w_spec = pl.BlockSpec((D, D), lambda i: (0, 0))

_tc_prep = pl.pallas_call(
    _tc_prep_body,
    grid=(TGRID,),
    in_specs=[_row_spec, _col_spec, _w_spec],
    out_specs=_row_spec,
    out_shape=jax.ShapeDtypeStruct((NPAD, D), jnp.float32),
)

_tc_mid = pl.pallas_call(
    _tc_mid_body,
    grid=(TGRID,),
    in_specs=[_row_spec, _row_spec, _row_spec, _col_spec, _bias_spec, _w_spec],
    out_specs=_row_spec,
    out_shape=jax.ShapeDtypeStruct((NPAD, D), jnp.float32),
)

_tc_final = pl.pallas_call(
    _tc_final_body,
    grid=(TGRID,),
    in_specs=[_row_spec, _row_spec, _row_spec, _col_spec, _bias_spec],
    out_specs=_row_spec,
    out_shape=jax.ShapeDtypeStruct((N, D), jnp.float32),
)


def kernel(edge_index, x, W1, b1, W2, b2):
  src = edge_index[0]
  dst = edge_index[1]
  pad = jnp.full((EPAD - E,), N, dtype=jnp.int32)
  src3 = jnp.concatenate([src, pad]).reshape(NW, NHALF, CH_H, CH)
  dst3 = jnp.concatenate([dst, pad]).reshape(NW, NHALF, CH_H, CH)
  dst_deg = dst.reshape(NS, E_PT16)
  xpad = jnp.zeros((NPAD, D), jnp.float32).at[:N].set(x)
  b1r = b1.reshape(1, D)
  b2r = b2.reshape(1, D)

  dinv = _sc_deg(dst_deg).reshape(NPAD, 1)
  y1 = _tc_prep(xpad, dinv, W1)
  p10, p11 = _sc_edges(y1, src3, dst3)
  y2 = _tc_mid(p10, p11, y1, dinv, b1r, W2)
  p20, p21 = _sc_edges(y2, src3, dst3)
  return _tc_final(p20, p21, y2, dinv, b2r)
